# baseline (speedup 1.0000x reference)
import jax
import jax.numpy as jnp
from jax.experimental import pallas as pl

N = 10000
E = 320000
IN_DIM = 128
HID = 16
OUT = 16
H1 = 8


def _gat_layer(h, src, dst, W, attn_l, attn_r, num_heads, out_dim):
    feat = (h @ W).reshape(-1, num_heads, out_dim)
    el = (feat * attn_l[None, :, :]).sum(-1)
    er = (feat * attn_r[None, :, :]).sum(-1)
    e = el[src] + er[dst]
    e = jax.nn.leaky_relu(e, negative_slope=0.2)
    e_max = jax.ops.segment_max(e, dst, num_segments=N)
    e_max = jnp.where(jnp.isfinite(e_max), e_max, 0.0)
    e_exp = jnp.exp(e - e_max[dst])
    denom = jax.ops.segment_sum(e_exp, dst, num_segments=N)
    alpha = e_exp / (denom[dst] + 1e-9)
    msg = feat[src] * alpha[:, :, None]
    out = jax.ops.segment_sum(msg, dst, num_segments=N)
    return out.reshape(N, num_heads * out_dim)


def _elu_kernel(x_ref, o_ref):
    x = x_ref[...]
    o_ref[...] = jnp.where(x > 0, x, jnp.exp(x) - 1.0)


def _elu(x):
    n, d = x.shape
    return pl.pallas_call(
        _elu_kernel,
        out_shape=jax.ShapeDtypeStruct((n, d), x.dtype),
        grid=(n // 400,),
        in_specs=[pl.BlockSpec((400, d), lambda i: (i, 0))],
        out_specs=pl.BlockSpec((400, d), lambda i: (i, 0)),
    )(x)


def kernel(edge_index, h, snorm_n, snorm_e, W1, attn_l1, attn_r1, W2, attn_l2, attn_r2):
    src = edge_index[0]
    dst = edge_index[1]
    h1 = _elu(_gat_layer(h, src, dst, W1, attn_l1, attn_r1, H1, HID))
    h2 = _elu(_gat_layer(h1, src, dst, W2, attn_l2, attn_r2, 1, OUT))
    return h2


# speedup vs baseline: 41.6624x; 41.6624x over previous
import functools

import jax
import jax.numpy as jnp
from jax import lax
from jax.experimental import pallas as pl
from jax.experimental.pallas import tpu as pltpu
from jax.experimental.pallas import tpu_sc as plsc

N = 10000
E = 320000
IN_DIM = 128
HID = 16
OUT = 16
H1 = 8

NPAD = 10240
NCORES = 2
NSUB = 16
NW = NCORES * NSUB
EDGES_PER_TILE = E // NW
CH = 80
ROWS_PER_TILE = NPAD // NSUB
BLK = 400
GRID = N // BLK


def _tc_prep1(h_ref, w1_ref, alf_ref, arf_ref, f1_ref, er1_ref):
    feat = jnp.dot(h_ref[...], w1_ref[...], preferred_element_type=jnp.float32)
    elf = feat * alf_ref[...]
    erf = feat * arf_ref[...]
    r = lax.broadcasted_iota(jnp.int32, (H1 * HID, H1), 0)
    c = lax.broadcasted_iota(jnp.int32, (H1 * HID, H1), 1)
    sel = (r // HID == c).astype(jnp.float32)
    el = jnp.dot(elf, sel, preferred_element_type=jnp.float32)
    er = jnp.dot(erf, sel, preferred_element_type=jnp.float32)
    f1_ref[...] = jnp.concatenate([feat, el, er], axis=1)
    er1_ref[...] = jnp.concatenate(
        [er, jnp.zeros((BLK, 8), jnp.float32)], axis=1)


def _tc_mid(pa_ref, pb_ref, w2_ref, al2_ref, ar2_ref, f2_ref, er2_ref):
    p = pa_ref[...] + pb_ref[...]
    blocks = []
    for hh in range(H1):
        num = p[:, hh * HID:(hh + 1) * HID]
        den = p[:, 128 + hh:129 + hh] + 1e-9
        blocks.append(num / den)
    x = jnp.concatenate(blocks, axis=1)
    h1 = jnp.where(x > 0, x, jnp.exp(x) - 1.0)
    feat2 = jnp.dot(h1, w2_ref[...], preferred_element_type=jnp.float32)
    el2 = jnp.dot(feat2, al2_ref[...], preferred_element_type=jnp.float32)
    er2 = jnp.dot(feat2, ar2_ref[...], preferred_element_type=jnp.float32)
    zpad = jnp.zeros((BLK, 15), jnp.float32)
    f2_ref[...] = jnp.concatenate([feat2, el2, zpad], axis=1)
    er2_ref[...] = jnp.concatenate([er2, zpad], axis=1)


def _tc_fin(pa_ref, pb_ref, o_ref):
    p = pa_ref[...] + pb_ref[...]
    x = p[:, 0:OUT] / (p[:, OUT:OUT + 1] + 1e-9)
    o_ref[...] = jnp.where(x > 0, x, jnp.exp(x) - 1.0)


def _make_edge_pass(p_row, att_off, n_heads):
    chunks = EDGES_PER_TILE // CH
    copies = ROWS_PER_TILE // CH

    def body(src_ref, dst_ref, f_ref, er_ref, out_ref,
             src_v, dst_v, fs_v, er_v, o_v, acc, sem):
        c = lax.axis_index("c")
        s = lax.axis_index("s")
        wid = c * NSUB + s
        zero16 = jnp.zeros((16,), jnp.float32)

        def zrow(i, carry):
            for j in range(p_row // 16):
                o_v[i, pl.ds(j * 16, 16)] = zero16
            return carry
        lax.fori_loop(0, CH, zrow, 0)
        for k in range(copies):
            pltpu.sync_copy(o_v, acc.at[pl.ds(s * ROWS_PER_TILE + k * CH, CH)])
        plsc.subcore_barrier()

        def chunk_body(k, carry):
            base = wid * EDGES_PER_TILE + k * CH
            pltpu.sync_copy(src_ref.at[pl.ds(base, CH)], src_v)
            pltpu.sync_copy(dst_ref.at[pl.ds(base, CH)], dst_v)
            pltpu.async_copy(f_ref.at[src_v], fs_v, sem).wait()
            pltpu.async_copy(er_ref.at[dst_v], er_v, sem).wait()

            def edge_body(i, carry2):
                e = fs_v[i, pl.ds(att_off, 16)] + er_v[i, :]
                e = jnp.where(e > 0, e, 0.2 * e)
                w = jnp.exp(e)
                o_v[i, pl.ds(att_off, 16)] = w
                for hh in range(n_heads):
                    sw = w[hh]
                    o_v[i, pl.ds(hh * HID, HID)] = (
                        sw * fs_v[i, pl.ds(hh * HID, HID)])
                return carry2
            lax.fori_loop(0, CH, edge_body, 0)
            pltpu.sync_copy(o_v, acc.at[dst_v], add=True)
            return carry
        lax.fori_loop(0, chunks, chunk_body, 0)
        plsc.subcore_barrier()

        for k in range(copies):
            r0 = s * ROWS_PER_TILE + k * CH
            pltpu.sync_copy(acc.at[pl.ds(r0, CH)], o_v)
            pltpu.sync_copy(o_v, out_ref.at[c, pl.ds(r0, CH)])

    mesh = plsc.VectorSubcoreMesh(core_axis_name="c", subcore_axis_name="s")
    return pl.kernel(
        body,
        out_type=jax.ShapeDtypeStruct((NCORES, NPAD, p_row), jnp.float32),
        mesh=mesh,
        compiler_params=pltpu.CompilerParams(use_tc_tiling_on_sc=False),
        scratch_types=[
            pltpu.VMEM((CH,), jnp.int32),
            pltpu.VMEM((CH,), jnp.int32),
            pltpu.VMEM((CH, p_row), jnp.float32),
            pltpu.VMEM((CH, 16), jnp.float32),
            pltpu.VMEM((CH, p_row), jnp.float32),
            pltpu.VMEM_SHARED((NPAD, p_row), jnp.float32),
            pltpu.SemaphoreType.DMA,
        ],
    )


_edge_pass_1 = _make_edge_pass(144, 128, H1)
_edge_pass_2 = _make_edge_pass(32, 16, 1)


def kernel(edge_index, h, snorm_n, snorm_e, W1, attn_l1, attn_r1,
           W2, attn_l2, attn_r2):
    src = edge_index[0]
    dst = edge_index[1]
    alf = attn_l1.reshape(1, H1 * HID)
    arf = attn_r1.reshape(1, H1 * HID)
    F1, ER1 = pl.pallas_call(
        _tc_prep1,
        grid=(GRID,),
        in_specs=[
            pl.BlockSpec((BLK, IN_DIM), lambda i: (i, 0)),
            pl.BlockSpec((IN_DIM, H1 * HID), lambda i: (0, 0)),
            pl.BlockSpec((1, H1 * HID), lambda i: (0, 0)),
            pl.BlockSpec((1, H1 * HID), lambda i: (0, 0)),
        ],
        out_specs=[
            pl.BlockSpec((BLK, 144), lambda i: (i, 0)),
            pl.BlockSpec((BLK, 16), lambda i: (i, 0)),
        ],
        out_shape=[
            jax.ShapeDtypeStruct((N, 144), jnp.float32),
            jax.ShapeDtypeStruct((N, 16), jnp.float32),
        ],
    )(h, W1, alf, arf)

    P1 = _edge_pass_1(src, dst, F1, ER1)

    F2, ER2 = pl.pallas_call(
        _tc_mid,
        grid=(GRID,),
        in_specs=[
            pl.BlockSpec((BLK, 144), lambda i: (i, 0)),
            pl.BlockSpec((BLK, 144), lambda i: (i, 0)),
            pl.BlockSpec((H1 * HID, OUT), lambda i: (0, 0)),
            pl.BlockSpec((OUT, 1), lambda i: (0, 0)),
            pl.BlockSpec((OUT, 1), lambda i: (0, 0)),
        ],
        out_specs=[
            pl.BlockSpec((BLK, 32), lambda i: (i, 0)),
            pl.BlockSpec((BLK, 16), lambda i: (i, 0)),
        ],
        out_shape=[
            jax.ShapeDtypeStruct((N, 32), jnp.float32),
            jax.ShapeDtypeStruct((N, 16), jnp.float32),
        ],
    )(P1[0, :N], P1[1, :N], W2, attn_l2.reshape(OUT, 1),
      attn_r2.reshape(OUT, 1))

    P2 = _edge_pass_2(src, dst, F2, ER2)

    out = pl.pallas_call(
        _tc_fin,
        grid=(GRID,),
        in_specs=[
            pl.BlockSpec((BLK, 32), lambda i: (i, 0)),
            pl.BlockSpec((BLK, 32), lambda i: (i, 0)),
        ],
        out_specs=pl.BlockSpec((BLK, OUT), lambda i: (i, 0)),
        out_shape=jax.ShapeDtypeStruct((N, OUT), jnp.float32),
    )(P2[0, :N], P2[1, :N])
    return out


# speedup vs baseline: 52.5689x; 1.2618x over previous
import jax
import jax.numpy as jnp
from jax import lax
from jax.experimental import pallas as pl
from jax.experimental.pallas import tpu as pltpu
from jax.experimental.pallas import tpu_sc as plsc

N = 10000
E = 320000
IN_DIM = 128
HID = 16
OUT = 16
H1 = 8

NPAD = 10240
NCORES = 2
NSUB = 16
NW = NCORES * NSUB
CH = 128
EPT = 10240
CHUNKS = EPT // CH
EP = NW * EPT
NB = 2
GROUPS = CHUNKS // NB
ROWS_PER_TILE = NPAD // NSUB
COPIES = ROWS_PER_TILE // CH
BLK = 512
GRID = NPAD // BLK


def _tc_prep1(h_ref, w1_ref, alf_ref, arf_ref,
              f1a_ref, er1a_ref, f1b_ref, er1b_ref):
    feat = jnp.dot(h_ref[...], w1_ref[...], preferred_element_type=jnp.float32)
    elf = feat * alf_ref[...]
    erf = feat * arf_ref[...]
    r = lax.broadcasted_iota(jnp.int32, (H1 * HID, H1), 0)
    c = lax.broadcasted_iota(jnp.int32, (H1 * HID, H1), 1)
    sel = (r // HID == c).astype(jnp.float32)
    el = jnp.dot(elf, sel, preferred_element_type=jnp.float32)
    er = jnp.dot(erf, sel, preferred_element_type=jnp.float32)
    z8 = jnp.zeros((BLK, 8), jnp.float32)
    z12 = jnp.zeros((BLK, 12), jnp.float32)
    f1a_ref[...] = jnp.concatenate(
        [feat[:, 0:64], el[:, 0:4], er[:, 0:4], z8], axis=1)
    er1a_ref[...] = jnp.concatenate([er[:, 0:4], z12], axis=1)
    f1b_ref[...] = jnp.concatenate(
        [feat[:, 64:128], el[:, 4:8], er[:, 4:8], z8], axis=1)
    er1b_ref[...] = jnp.concatenate([er[:, 4:8], z12], axis=1)


def _tc_mid(pa_ref, pb_ref, w2_ref, al2_ref, ar2_ref, f2_ref, er2_ref):
    pa = pa_ref[0] + pa_ref[1]
    pb = pb_ref[0] + pb_ref[1]
    blocks = []
    for hh in range(H1):
        p = pa if hh < 4 else pb
        j = hh % 4
        num = p[:, j * HID:(j + 1) * HID]
        den = p[:, 64 + j:65 + j] + 1e-9
        blocks.append(num / den)
    x = jnp.concatenate(blocks, axis=1)
    h1 = jnp.where(x > 0, x, jnp.exp(x) - 1.0)
    feat2 = jnp.dot(h1, w2_ref[...], preferred_element_type=jnp.float32)
    el2 = jnp.dot(feat2, al2_ref[...], preferred_element_type=jnp.float32)
    er2 = jnp.dot(feat2, ar2_ref[...], preferred_element_type=jnp.float32)
    zpad = jnp.zeros((BLK, 15), jnp.float32)
    f2_ref[...] = jnp.concatenate([feat2, el2, zpad], axis=1)
    er2_ref[...] = jnp.concatenate([er2, zpad], axis=1)


def _tc_fin(p_ref, o_ref):
    p = p_ref[0] + p_ref[1]
    x = p[:, 0:OUT] / (p[:, OUT:OUT + 1] + 1e-9)
    o_ref[...] = jnp.where(x > 0, x, jnp.exp(x) - 1.0)


def _make_edge_pass(p_row, att_off, n_heads):
    def body(src_ref, dst_ref, f_ref, er_ref, out_ref,
             src2d, dst2d, dst_s, fs_l, er_l, o_l, acc,
             semf, seme, sems, semi):
        c = lax.axis_index("c")
        s = lax.axis_index("s")
        wid = c * NSUB + s
        zero16 = jnp.zeros((16,), jnp.float32)

        pltpu.sync_copy(src_ref.at[wid], src2d)
        pltpu.sync_copy(dst_ref.at[wid], dst2d)

        def zrow(i, carry):
            for j in range(p_row // 16):
                o_l[0][i, pl.ds(j * 16, 16)] = zero16
            return carry
        lax.fori_loop(0, CH, zrow, 0)
        for k in range(COPIES):
            pltpu.sync_copy(o_l[0],
                            acc.at[pl.ds(s * ROWS_PER_TILE + k * CH, CH)])
        plsc.subcore_barrier()

        def gstart(k, b):
            pltpu.async_copy(f_ref.at[src2d.at[k]], fs_l[b], semf[b])
            pltpu.async_copy(er_ref.at[dst2d.at[k]], er_l[b], seme[b])

        def gwait(b):
            pltpu.make_async_copy(f_ref.at[src2d.at[0]], fs_l[b],
                                  semf[b]).wait()
            pltpu.make_async_copy(er_ref.at[dst2d.at[0]], er_l[b],
                                  seme[b]).wait()

        def dsstart(k, b):
            pltpu.async_copy(dst_ref.at[wid, k], dst_s[b], semi[b])

        def dswait(b):
            pltpu.make_async_copy(dst_ref.at[wid, 0], dst_s[b],
                                  semi[b]).wait()

        def sstart(b):
            pltpu.async_copy(o_l[b], acc.at[dst_s[b]], sems[b], add=True)

        def swait(b):
            pltpu.make_async_copy(o_l[b], acc.at[dst_s[b]],
                                  sems[b]).wait()

        def compute(b):
            def edge_body(i, carry2):
                e = fs_l[b][i, pl.ds(att_off, 16)] + er_l[b][i, :]
                e = jnp.where(e > 0, e, 0.2 * e)
                w = jnp.exp(e)
                o_l[b][i, pl.ds(att_off, 16)] = w
                for hh in range(n_heads):
                    sw = w[hh]
                    o_l[b][i, pl.ds(hh * HID, HID)] = (
                        sw * fs_l[b][i, pl.ds(hh * HID, HID)])
                return carry2
            lax.fori_loop(0, CH, edge_body, 0)

        for b in range(NB):
            gstart(b, b)
            dsstart(b, b)
        for b in range(NB):
            gwait(b)
            compute(b)
            dswait(b)
            sstart(b)
            gstart(NB + b, b)

        def group(g, carry):
            for b in range(NB):
                k = g * NB + b
                gwait(b)
                swait(b)
                dsstart(k, b)
                compute(b)
                dswait(b)
                sstart(b)
                gstart(k + NB, b)
            return carry
        lax.fori_loop(1, GROUPS - 1, group, 0)
        for b in range(NB):
            k = (GROUPS - 1) * NB + b
            gwait(b)
            swait(b)
            dsstart(k, b)
            compute(b)
            dswait(b)
            sstart(b)
        for b in range(NB):
            swait(b)
        plsc.subcore_barrier()

        for k in range(COPIES):
            r0 = s * ROWS_PER_TILE + k * CH
            pltpu.sync_copy(acc.at[pl.ds(r0, CH)], o_l[0])
            pltpu.sync_copy(o_l[0], out_ref.at[c, pl.ds(r0, CH)])

    mesh = plsc.VectorSubcoreMesh(core_axis_name="c", subcore_axis_name="s")
    return pl.kernel(
        body,
        out_type=jax.ShapeDtypeStruct((NCORES, NPAD, p_row), jnp.float32),
        mesh=mesh,
        compiler_params=pltpu.CompilerParams(use_tc_tiling_on_sc=False),
        scratch_types=[
            pltpu.VMEM((CHUNKS, CH), jnp.int32),
            pltpu.VMEM((CHUNKS, CH), jnp.int32),
            [pltpu.VMEM((CH,), jnp.int32)] * NB,
            [pltpu.VMEM((CH, p_row), jnp.float32)] * NB,
            [pltpu.VMEM((CH, 16), jnp.float32)] * NB,
            [pltpu.VMEM((CH, p_row), jnp.float32)] * NB,
            pltpu.VMEM_SHARED((NPAD, p_row), jnp.float32),
            [pltpu.SemaphoreType.DMA] * NB,
            [pltpu.SemaphoreType.DMA] * NB,
            [pltpu.SemaphoreType.DMA] * NB,
            [pltpu.SemaphoreType.DMA] * NB,
        ],
    )


_edge_pass_h4 = _make_edge_pass(80, 64, 4)
_edge_pass_2 = _make_edge_pass(32, 16, 1)


def kernel(edge_index, h, snorm_n, snorm_e, W1, attn_l1, attn_r1,
           W2, attn_l2, attn_r2):
    pad = jnp.full((EP - E,), N, jnp.int32)
    src = jnp.concatenate([edge_index[0], pad]).reshape(NW, CHUNKS, CH)
    dst = jnp.concatenate([edge_index[1], pad]).reshape(NW, CHUNKS, CH)
    hp = jnp.pad(h, ((0, NPAD - N), (0, 0)))
    alf = attn_l1.reshape(1, H1 * HID)
    arf = attn_r1.reshape(1, H1 * HID)
    F1a, ER1a, F1b, ER1b = pl.pallas_call(
        _tc_prep1,
        grid=(GRID,),
        in_specs=[
            pl.BlockSpec((BLK, IN_DIM), lambda i: (i, 0)),
            pl.BlockSpec((IN_DIM, H1 * HID), lambda i: (0, 0)),
            pl.BlockSpec((1, H1 * HID), lambda i: (0, 0)),
            pl.BlockSpec((1, H1 * HID), lambda i: (0, 0)),
        ],
        out_specs=[
            pl.BlockSpec((BLK, 80), lambda i: (i, 0)),
            pl.BlockSpec((BLK, 16), lambda i: (i, 0)),
            pl.BlockSpec((BLK, 80), lambda i: (i, 0)),
            pl.BlockSpec((BLK, 16), lambda i: (i, 0)),
        ],
        out_shape=[
            jax.ShapeDtypeStruct((NPAD, 80), jnp.float32),
            jax.ShapeDtypeStruct((NPAD, 16), jnp.float32),
            jax.ShapeDtypeStruct((NPAD, 80), jnp.float32),
            jax.ShapeDtypeStruct((NPAD, 16), jnp.float32),
        ],
    )(hp, W1, alf, arf)

    P1a = _edge_pass_h4(src, dst, F1a, ER1a)
    P1b = _edge_pass_h4(src, dst, F1b, ER1b)

    F2, ER2 = pl.pallas_call(
        _tc_mid,
        grid=(GRID,),
        in_specs=[
            pl.BlockSpec((NCORES, BLK, 80), lambda i: (0, i, 0)),
            pl.BlockSpec((NCORES, BLK, 80), lambda i: (0, i, 0)),
            pl.BlockSpec((H1 * HID, OUT), lambda i: (0, 0)),
            pl.BlockSpec((OUT, 1), lambda i: (0, 0)),
            pl.BlockSpec((OUT, 1), lambda i: (0, 0)),
        ],
        out_specs=[
            pl.BlockSpec((BLK, 32), lambda i: (i, 0)),
            pl.BlockSpec((BLK, 16), lambda i: (i, 0)),
        ],
        out_shape=[
            jax.ShapeDtypeStruct((NPAD, 32), jnp.float32),
            jax.ShapeDtypeStruct((NPAD, 16), jnp.float32),
        ],
    )(P1a, P1b, W2, attn_l2.reshape(OUT, 1), attn_r2.reshape(OUT, 1))

    P2 = _edge_pass_2(src, dst, F2, ER2)

    out = pl.pallas_call(
        _tc_fin,
        grid=(GRID,),
        in_specs=[pl.BlockSpec((NCORES, BLK, 32), lambda i: (0, i, 0))],
        out_specs=pl.BlockSpec((BLK, OUT), lambda i: (i, 0)),
        out_shape=jax.ShapeDtypeStruct((NPAD, OUT), jnp.float32),
    )(P2)
    return out[:N]


# speedup vs baseline: 52.6737x; 1.0020x over previous
import jax
import jax.numpy as jnp
from jax import lax
from jax.experimental import pallas as pl
from jax.experimental.pallas import tpu as pltpu
from jax.experimental.pallas import tpu_sc as plsc

N = 10000
E = 320000
IN_DIM = 128
HID = 16
OUT = 16
H1 = 8

NPAD = 10240
NCORES = 2
NSUB = 16
NW = NCORES * NSUB
CH = 128
EPT = 10240
CHUNKS = EPT // CH
EP = NW * EPT
NB = 2
GROUPS = CHUNKS // NB
ROWS_PER_TILE = NPAD // NSUB
COPIES = ROWS_PER_TILE // CH
BLK = 512
GRID = NPAD // BLK


def _tc_prep1(h_ref, w1_ref, alf_ref, arf_ref,
              f1a_ref, er1a_ref, f1b_ref, er1b_ref):
    feat = jnp.dot(h_ref[...], w1_ref[...], preferred_element_type=jnp.float32)
    elf = feat * alf_ref[...]
    erf = feat * arf_ref[...]
    r = lax.broadcasted_iota(jnp.int32, (H1 * HID, H1), 0)
    c = lax.broadcasted_iota(jnp.int32, (H1 * HID, H1), 1)
    sel = (r // HID == c).astype(jnp.float32)
    el = jnp.dot(elf, sel, preferred_element_type=jnp.float32)
    er = jnp.dot(erf, sel, preferred_element_type=jnp.float32)
    z8 = jnp.zeros((BLK, 8), jnp.float32)
    z12 = jnp.zeros((BLK, 12), jnp.float32)
    f1a_ref[...] = jnp.concatenate(
        [feat[:, 0:64], el[:, 0:4], er[:, 0:4], z8], axis=1)
    er1a_ref[...] = jnp.concatenate([er[:, 0:4], z12], axis=1)
    f1b_ref[...] = jnp.concatenate(
        [feat[:, 64:128], el[:, 4:8], er[:, 4:8], z8], axis=1)
    er1b_ref[...] = jnp.concatenate([er[:, 4:8], z12], axis=1)


def _tc_mid(pa_ref, pb_ref, w2_ref, al2_ref, ar2_ref, f2_ref, er2_ref):
    pa = pa_ref[0] + pa_ref[1]
    pb = pb_ref[0] + pb_ref[1]
    blocks = []
    for hh in range(H1):
        p = pa if hh < 4 else pb
        j = hh % 4
        num = p[:, j * HID:(j + 1) * HID]
        den = p[:, 64 + j:65 + j] + 1e-9
        blocks.append(num / den)
    x = jnp.concatenate(blocks, axis=1)
    h1 = jnp.where(x > 0, x, jnp.exp(x) - 1.0)
    feat2 = jnp.dot(h1, w2_ref[...], preferred_element_type=jnp.float32)
    el2 = jnp.dot(feat2, al2_ref[...], preferred_element_type=jnp.float32)
    er2 = jnp.dot(feat2, ar2_ref[...], preferred_element_type=jnp.float32)
    zpad = jnp.zeros((BLK, 15), jnp.float32)
    f2_ref[...] = jnp.concatenate([feat2, el2, zpad], axis=1)
    er2_ref[...] = jnp.concatenate([er2, zpad], axis=1)


def _tc_fin(p_ref, o_ref):
    p = p_ref[0] + p_ref[1]
    x = p[:, 0:OUT] / (p[:, OUT:OUT + 1] + 1e-9)
    o_ref[...] = jnp.where(x > 0, x, jnp.exp(x) - 1.0)


def _make_edge_pass(p_row, att_off, n_heads):
    def body(src_ref, dst_ref, f_ref, er_ref, out_ref,
             src2d, dst2d, dst_s, fs_l, er_l, o_l, acc,
             semf, seme, sems, semi):
        c = lax.axis_index("c")
        s = lax.axis_index("s")
        wid = c * NSUB + s
        zero16 = jnp.zeros((16,), jnp.float32)

        pltpu.sync_copy(src_ref.at[wid], src2d)
        pltpu.sync_copy(dst_ref.at[wid], dst2d)

        def zrow(i, carry):
            for j in range(p_row // 16):
                o_l[0][i, pl.ds(j * 16, 16)] = zero16
            return carry
        lax.fori_loop(0, CH, zrow, 0)
        for k in range(COPIES):
            pltpu.sync_copy(o_l[0],
                            acc.at[pl.ds(s * ROWS_PER_TILE + k * CH, CH)])
        plsc.subcore_barrier()

        def gstart(k, b):
            pltpu.async_copy(f_ref.at[src2d.at[k]], fs_l[b], semf[b])
            pltpu.async_copy(er_ref.at[dst2d.at[k]], er_l[b], seme[b])

        def gwait(b):
            pltpu.make_async_copy(f_ref.at[src2d.at[0]], fs_l[b],
                                  semf[b]).wait()
            pltpu.make_async_copy(er_ref.at[dst2d.at[0]], er_l[b],
                                  seme[b]).wait()

        def dsstart(k, b):
            pltpu.async_copy(dst_ref.at[wid, k], dst_s[b], semi[b])

        def dswait(b):
            pltpu.make_async_copy(dst_ref.at[wid, 0], dst_s[b],
                                  semi[b]).wait()

        def sstart(b):
            pltpu.async_copy(o_l[b], acc.at[dst_s[b]], sems[b], add=True)

        def swait(b):
            pltpu.make_async_copy(o_l[b], acc.at[dst_s[b]],
                                  sems[b]).wait()

        iota16 = lax.iota(jnp.int32, 16)

        def compute(b):
            fs, er, o = fs_l[b], er_l[b], o_l[b]

            def blk_body(t, carry2):
                i0 = t * 16
                rows = i0 + iota16
                ws = []
                for hh in range(n_heads):
                    el16 = plsc.load_gather(
                        fs, [rows, jnp.full((16,), att_off + hh, jnp.int32)])
                    er16 = plsc.load_gather(
                        er, [rows, jnp.full((16,), hh, jnp.int32)])
                    e = el16 + er16
                    e = jnp.where(e > 0, e, 0.2 * e)
                    w = jnp.exp(e)
                    plsc.store_scatter(
                        o, [rows, jnp.full((16,), att_off + hh, jnp.int32)],
                        w)
                    ws.append(w)
                for j in range(16):
                    for hh in range(n_heads):
                        swv = lax.gather(
                            ws[hh], jnp.full((16, 1), j, jnp.int32),
                            lax.GatherDimensionNumbers(
                                offset_dims=(), collapsed_slice_dims=(0,),
                                start_index_map=(0,)),
                            (1,),
                            mode=lax.GatherScatterMode.PROMISE_IN_BOUNDS)
                        o[i0 + j, pl.ds(hh * HID, HID)] = (
                            swv * fs[i0 + j, pl.ds(hh * HID, HID)])
                return carry2
            lax.fori_loop(0, CH // 16, blk_body, 0)

        for b in range(NB):
            gstart(b, b)
            dsstart(b, b)
        for b in range(NB):
            gwait(b)
            compute(b)
            dswait(b)
            sstart(b)
            gstart(NB + b, b)

        def group(g, carry):
            for b in range(NB):
                k = g * NB + b
                gwait(b)
                swait(b)
                dsstart(k, b)
                compute(b)
                dswait(b)
                sstart(b)
                gstart(k + NB, b)
            return carry
        lax.fori_loop(1, GROUPS - 1, group, 0)
        for b in range(NB):
            k = (GROUPS - 1) * NB + b
            gwait(b)
            swait(b)
            dsstart(k, b)
            compute(b)
            dswait(b)
            sstart(b)
        for b in range(NB):
            swait(b)
        plsc.subcore_barrier()

        for k in range(COPIES):
            r0 = s * ROWS_PER_TILE + k * CH
            pltpu.sync_copy(acc.at[pl.ds(r0, CH)], o_l[0])
            pltpu.sync_copy(o_l[0], out_ref.at[c, pl.ds(r0, CH)])

    mesh = plsc.VectorSubcoreMesh(core_axis_name="c", subcore_axis_name="s")
    return pl.kernel(
        body,
        out_type=jax.ShapeDtypeStruct((NCORES, NPAD, p_row), jnp.float32),
        mesh=mesh,
        compiler_params=pltpu.CompilerParams(use_tc_tiling_on_sc=False,
                                             needs_layout_passes=False),
        scratch_types=[
            pltpu.VMEM((CHUNKS, CH), jnp.int32),
            pltpu.VMEM((CHUNKS, CH), jnp.int32),
            [pltpu.VMEM((CH,), jnp.int32)] * NB,
            [pltpu.VMEM((CH, p_row), jnp.float32)] * NB,
            [pltpu.VMEM((CH, 16), jnp.float32)] * NB,
            [pltpu.VMEM((CH, p_row), jnp.float32)] * NB,
            pltpu.VMEM_SHARED((NPAD, p_row), jnp.float32),
            [pltpu.SemaphoreType.DMA] * NB,
            [pltpu.SemaphoreType.DMA] * NB,
            [pltpu.SemaphoreType.DMA] * NB,
            [pltpu.SemaphoreType.DMA] * NB,
        ],
    )


_edge_pass_h4 = _make_edge_pass(80, 64, 4)
_edge_pass_2 = _make_edge_pass(32, 16, 1)


def kernel(edge_index, h, snorm_n, snorm_e, W1, attn_l1, attn_r1,
           W2, attn_l2, attn_r2):
    pad = jnp.full((EP - E,), N, jnp.int32)
    src = jnp.concatenate([edge_index[0], pad]).reshape(NW, CHUNKS, CH)
    dst = jnp.concatenate([edge_index[1], pad]).reshape(NW, CHUNKS, CH)
    hp = jnp.pad(h, ((0, NPAD - N), (0, 0)))
    alf = attn_l1.reshape(1, H1 * HID)
    arf = attn_r1.reshape(1, H1 * HID)
    F1a, ER1a, F1b, ER1b = pl.pallas_call(
        _tc_prep1,
        grid=(GRID,),
        in_specs=[
            pl.BlockSpec((BLK, IN_DIM), lambda i: (i, 0)),
            pl.BlockSpec((IN_DIM, H1 * HID), lambda i: (0, 0)),
            pl.BlockSpec((1, H1 * HID), lambda i: (0, 0)),
            pl.BlockSpec((1, H1 * HID), lambda i: (0, 0)),
        ],
        out_specs=[
            pl.BlockSpec((BLK, 80), lambda i: (i, 0)),
            pl.BlockSpec((BLK, 16), lambda i: (i, 0)),
            pl.BlockSpec((BLK, 80), lambda i: (i, 0)),
            pl.BlockSpec((BLK, 16), lambda i: (i, 0)),
        ],
        out_shape=[
            jax.ShapeDtypeStruct((NPAD, 80), jnp.float32),
            jax.ShapeDtypeStruct((NPAD, 16), jnp.float32),
            jax.ShapeDtypeStruct((NPAD, 80), jnp.float32),
            jax.ShapeDtypeStruct((NPAD, 16), jnp.float32),
        ],
    )(hp, W1, alf, arf)

    P1a = _edge_pass_h4(src, dst, F1a, ER1a)
    P1b = _edge_pass_h4(src, dst, F1b, ER1b)

    F2, ER2 = pl.pallas_call(
        _tc_mid,
        grid=(GRID,),
        in_specs=[
            pl.BlockSpec((NCORES, BLK, 80), lambda i: (0, i, 0)),
            pl.BlockSpec((NCORES, BLK, 80), lambda i: (0, i, 0)),
            pl.BlockSpec((H1 * HID, OUT), lambda i: (0, 0)),
            pl.BlockSpec((OUT, 1), lambda i: (0, 0)),
            pl.BlockSpec((OUT, 1), lambda i: (0, 0)),
        ],
        out_specs=[
            pl.BlockSpec((BLK, 32), lambda i: (i, 0)),
            pl.BlockSpec((BLK, 16), lambda i: (i, 0)),
        ],
        out_shape=[
            jax.ShapeDtypeStruct((NPAD, 32), jnp.float32),
            jax.ShapeDtypeStruct((NPAD, 16), jnp.float32),
        ],
    )(P1a, P1b, W2, attn_l2.reshape(OUT, 1), attn_r2.reshape(OUT, 1))

    P2 = _edge_pass_2(src, dst, F2, ER2)

    out = pl.pallas_call(
        _tc_fin,
        grid=(GRID,),
        in_specs=[pl.BlockSpec((NCORES, BLK, 32), lambda i: (0, i, 0))],
        out_specs=pl.BlockSpec((BLK, OUT), lambda i: (i, 0)),
        out_shape=jax.ShapeDtypeStruct((NPAD, OUT), jnp.float32),
    )(P2)
    return out[:N]


# speedup vs baseline: 53.5110x; 1.0159x over previous
import jax
import jax.numpy as jnp
from jax import lax
from jax.experimental import pallas as pl
from jax.experimental.pallas import tpu as pltpu
from jax.experimental.pallas import tpu_sc as plsc

N = 10000
E = 320000
IN_DIM = 128
HID = 16
OUT = 16
H1 = 8

NPAD = 10240
NCORES = 2
NSUB = 16
NW = NCORES * NSUB
CH = 128
EPT = 10240
CHUNKS = EPT // CH
EP = NW * EPT
NB = 2
GROUPS = CHUNKS // NB
ROWS_PER_TILE = NPAD // NSUB
COPIES = ROWS_PER_TILE // CH
BLK = 512
GRID = NPAD // BLK


def _tc_prep1(h_ref, w1_ref, alf_ref, arf_ref,
              f1a_ref, er1a_ref, f1b_ref, er1b_ref):
    feat = jnp.dot(h_ref[...], w1_ref[...], preferred_element_type=jnp.float32)
    elf = feat * alf_ref[...]
    erf = feat * arf_ref[...]
    r = lax.broadcasted_iota(jnp.int32, (H1 * HID, H1), 0)
    c = lax.broadcasted_iota(jnp.int32, (H1 * HID, H1), 1)
    sel = (r // HID == c).astype(jnp.float32)
    el = jnp.dot(elf, sel, preferred_element_type=jnp.float32)
    er = jnp.dot(erf, sel, preferred_element_type=jnp.float32)
    z8 = jnp.zeros((BLK, 8), jnp.float32)
    z12 = jnp.zeros((BLK, 12), jnp.float32)
    f1a_ref[...] = jnp.concatenate(
        [feat[:, 0:64], el[:, 0:4], er[:, 0:4], z8], axis=1)
    er1a_ref[...] = jnp.concatenate([er[:, 0:4], z12], axis=1)
    f1b_ref[...] = jnp.concatenate(
        [feat[:, 64:128], el[:, 4:8], er[:, 4:8], z8], axis=1)
    er1b_ref[...] = jnp.concatenate([er[:, 4:8], z12], axis=1)


def _tc_mid(p_ref, w2_ref, al2_ref, ar2_ref, f2_ref, er2_ref):
    pa = p_ref[0]
    pb = p_ref[1]
    blocks = []
    for hh in range(H1):
        p = pa if hh < 4 else pb
        j = hh % 4
        num = p[:, j * HID:(j + 1) * HID]
        den = p[:, 64 + j:65 + j] + 1e-9
        blocks.append(num / den)
    x = jnp.concatenate(blocks, axis=1)
    h1 = jnp.where(x > 0, x, jnp.exp(x) - 1.0)
    feat2 = jnp.dot(h1, w2_ref[...], preferred_element_type=jnp.float32)
    el2 = jnp.dot(feat2, al2_ref[...], preferred_element_type=jnp.float32)
    er2 = jnp.dot(feat2, ar2_ref[...], preferred_element_type=jnp.float32)
    zpad = jnp.zeros((BLK, 15), jnp.float32)
    f2_ref[...] = jnp.concatenate([feat2, el2, zpad], axis=1)
    er2_ref[...] = jnp.concatenate([er2, zpad], axis=1)


def _tc_fin(p_ref, o_ref):
    p = p_ref[0] + p_ref[1]
    x = p[:, 0:OUT] / (p[:, OUT:OUT + 1] + 1e-9)
    o_ref[...] = jnp.where(x > 0, x, jnp.exp(x) - 1.0)


def _make_edge_pass(p_row, att_off, n_heads):
    def body(src_ref, dst_ref, f_ref, er_ref, out_ref,
             src2d, dst2d, dst_s, fs_l, er_l, o_l, acc,
             semf, seme, sems, semi):
        c = lax.axis_index("c")
        s = lax.axis_index("s")
        wid = c * NSUB + s
        zero16 = jnp.zeros((16,), jnp.float32)

        pltpu.sync_copy(src_ref.at[wid], src2d)
        pltpu.sync_copy(dst_ref.at[wid], dst2d)

        def zrow(i, carry):
            for j in range(p_row // 16):
                o_l[0][i, pl.ds(j * 16, 16)] = zero16
            return carry
        lax.fori_loop(0, CH, zrow, 0)
        for k in range(COPIES):
            pltpu.sync_copy(o_l[0],
                            acc.at[pl.ds(s * ROWS_PER_TILE + k * CH, CH)])
        plsc.subcore_barrier()

        def gstart(k, b):
            pltpu.async_copy(f_ref.at[src2d.at[k]], fs_l[b], semf[b])
            pltpu.async_copy(er_ref.at[dst2d.at[k]], er_l[b], seme[b])

        def gwait(b):
            pltpu.make_async_copy(f_ref.at[src2d.at[0]], fs_l[b],
                                  semf[b]).wait()
            pltpu.make_async_copy(er_ref.at[dst2d.at[0]], er_l[b],
                                  seme[b]).wait()

        def dsstart(k, b):
            pltpu.async_copy(dst_ref.at[wid, k], dst_s[b], semi[b])

        def dswait(b):
            pltpu.make_async_copy(dst_ref.at[wid, 0], dst_s[b],
                                  semi[b]).wait()

        def sstart(b):
            pltpu.async_copy(o_l[b], acc.at[dst_s[b]], sems[b], add=True)

        def swait(b):
            pltpu.make_async_copy(o_l[b], acc.at[dst_s[b]],
                                  sems[b]).wait()

        iota16 = lax.iota(jnp.int32, 16)

        def compute(b):
            fs, er, o = fs_l[b], er_l[b], o_l[b]

            def blk_body(t, carry2):
                i0 = t * 16
                rows = i0 + iota16
                ws = []
                for hh in range(n_heads):
                    el16 = plsc.load_gather(
                        fs, [rows, jnp.full((16,), att_off + hh, jnp.int32)])
                    er16 = plsc.load_gather(
                        er, [rows, jnp.full((16,), hh, jnp.int32)])
                    e = el16 + er16
                    e = jnp.where(e > 0, e, 0.2 * e)
                    w = jnp.exp(e)
                    plsc.store_scatter(
                        o, [rows, jnp.full((16,), att_off + hh, jnp.int32)],
                        w)
                    ws.append(w)
                for j in range(16):
                    for hh in range(n_heads):
                        swv = lax.gather(
                            ws[hh], jnp.full((16, 1), j, jnp.int32),
                            lax.GatherDimensionNumbers(
                                offset_dims=(), collapsed_slice_dims=(0,),
                                start_index_map=(0,)),
                            (1,),
                            mode=lax.GatherScatterMode.PROMISE_IN_BOUNDS)
                        o[i0 + j, pl.ds(hh * HID, HID)] = (
                            swv * fs[i0 + j, pl.ds(hh * HID, HID)])
                return carry2
            lax.fori_loop(0, CH // 16, blk_body, 0)

        for b in range(NB):
            gstart(b, b)
            dsstart(b, b)
        for b in range(NB):
            gwait(b)
            compute(b)
            dswait(b)
            sstart(b)
            gstart(NB + b, b)

        def group(g, carry):
            for b in range(NB):
                k = g * NB + b
                gwait(b)
                swait(b)
                dsstart(k, b)
                compute(b)
                dswait(b)
                sstart(b)
                gstart(k + NB, b)
            return carry
        lax.fori_loop(1, GROUPS - 1, group, 0)
        for b in range(NB):
            k = (GROUPS - 1) * NB + b
            gwait(b)
            swait(b)
            dsstart(k, b)
            compute(b)
            dswait(b)
            sstart(b)
        for b in range(NB):
            swait(b)
        plsc.subcore_barrier()

        for k in range(COPIES):
            r0 = s * ROWS_PER_TILE + k * CH
            pltpu.sync_copy(acc.at[pl.ds(r0, CH)], o_l[0])
            pltpu.sync_copy(o_l[0], out_ref.at[c, pl.ds(r0, CH)])

    mesh = plsc.VectorSubcoreMesh(core_axis_name="c", subcore_axis_name="s")
    return pl.kernel(
        body,
        out_type=jax.ShapeDtypeStruct((NCORES, NPAD, p_row), jnp.float32),
        mesh=mesh,
        compiler_params=pltpu.CompilerParams(use_tc_tiling_on_sc=False,
                                             needs_layout_passes=False),
        scratch_types=[
            pltpu.VMEM((CHUNKS, CH), jnp.int32),
            pltpu.VMEM((CHUNKS, CH), jnp.int32),
            [pltpu.VMEM((CH,), jnp.int32)] * NB,
            [pltpu.VMEM((CH, p_row), jnp.float32)] * NB,
            [pltpu.VMEM((CH, 16), jnp.float32)] * NB,
            [pltpu.VMEM((CH, p_row), jnp.float32)] * NB,
            pltpu.VMEM_SHARED((NPAD, p_row), jnp.float32),
            [pltpu.SemaphoreType.DMA] * NB,
            [pltpu.SemaphoreType.DMA] * NB,
            [pltpu.SemaphoreType.DMA] * NB,
            [pltpu.SemaphoreType.DMA] * NB,
        ],
    )


CHUNKS2 = 2 * CHUNKS
GROUPS2 = CHUNKS2 // NB


def _make_edge_pass_l1():
    p_row, att_off, n_heads = 80, 64, 4

    def body(src_ref, dst_ref, f_ref, er_ref, out_ref,
             src_s, dst2d, dst_s, fs_l, er_l, o_l, acc,
             semf, seme, sems, semi, semss):
        c = lax.axis_index("c")
        s = lax.axis_index("s")
        base = c * NPAD
        zero16 = jnp.zeros((16,), jnp.float32)

        pltpu.sync_copy(dst_ref.at[s], dst2d)

        def offrow(k, carry):
            for j in range(CH // 16):
                dst2d[k, pl.ds(j * 16, 16)] = (
                    dst2d[k, pl.ds(j * 16, 16)] + base)
            return carry
        lax.fori_loop(0, CHUNKS2, offrow, 0)

        def zrow(i, carry):
            for j in range(p_row // 16):
                o_l[0][i, pl.ds(j * 16, 16)] = zero16
            return carry
        lax.fori_loop(0, CH, zrow, 0)
        for k in range(COPIES):
            pltpu.sync_copy(o_l[0],
                            acc.at[pl.ds(s * ROWS_PER_TILE + k * CH, CH)])
        plsc.subcore_barrier()

        def ssload(k, b):
            pltpu.async_copy(src_ref.at[s, k], src_s[b], semss[b])

        def sswait(b):
            pltpu.make_async_copy(src_ref.at[s, 0], src_s[b],
                                  semss[b]).wait()

        def offset_src(b):
            for j in range(CH // 16):
                src_s[b][pl.ds(j * 16, 16)] = (
                    src_s[b][pl.ds(j * 16, 16)] + base)

        def gstart(k, b):
            pltpu.async_copy(f_ref.at[src_s[b]], fs_l[b], semf[b])
            pltpu.async_copy(er_ref.at[dst2d.at[k]], er_l[b], seme[b])

        def gwait(b):
            pltpu.make_async_copy(f_ref.at[src_s[b]], fs_l[b],
                                  semf[b]).wait()
            pltpu.make_async_copy(er_ref.at[dst2d.at[0]], er_l[b],
                                  seme[b]).wait()

        def dsstart(k, b):
            pltpu.async_copy(dst_ref.at[s, k], dst_s[b], semi[b])

        def dswait(b):
            pltpu.make_async_copy(dst_ref.at[s, 0], dst_s[b],
                                  semi[b]).wait()

        def sstart(b):
            pltpu.async_copy(o_l[b], acc.at[dst_s[b]], sems[b], add=True)

        def swait(b):
            pltpu.make_async_copy(o_l[b], acc.at[dst_s[b]],
                                  sems[b]).wait()

        iota16 = lax.iota(jnp.int32, 16)

        def compute(b):
            fs, er, o = fs_l[b], er_l[b], o_l[b]

            def blk_body(t, carry2):
                i0 = t * 16
                rows = i0 + iota16
                ws = []
                for hh in range(n_heads):
                    el16 = plsc.load_gather(
                        fs, [rows, jnp.full((16,), att_off + hh, jnp.int32)])
                    er16 = plsc.load_gather(
                        er, [rows, jnp.full((16,), hh, jnp.int32)])
                    e = el16 + er16
                    e = jnp.where(e > 0, e, 0.2 * e)
                    w = jnp.exp(e)
                    plsc.store_scatter(
                        o, [rows, jnp.full((16,), att_off + hh, jnp.int32)],
                        w)
                    ws.append(w)
                for j in range(16):
                    for hh in range(n_heads):
                        swv = lax.gather(
                            ws[hh], jnp.full((16, 1), j, jnp.int32),
                            lax.GatherDimensionNumbers(
                                offset_dims=(), collapsed_slice_dims=(0,),
                                start_index_map=(0,)),
                            (1,),
                            mode=lax.GatherScatterMode.PROMISE_IN_BOUNDS)
                        o[i0 + j, pl.ds(hh * HID, HID)] = (
                            swv * fs[i0 + j, pl.ds(hh * HID, HID)])
                return carry2
            lax.fori_loop(0, CH // 16, blk_body, 0)

        for b in range(NB):
            ssload(b, b)
        for b in range(NB):
            sswait(b)
            offset_src(b)
            gstart(b, b)
        for b in range(NB):
            gwait(b)
            ssload(NB + b, b)
            dsstart(b, b)
            compute(b)
            dswait(b)
            sstart(b)
            sswait(b)
            offset_src(b)
            gstart(NB + b, b)

        def group(g, carry):
            for b in range(NB):
                k = g * NB + b
                gwait(b)
                ssload(k + NB, b)
                swait(b)
                dsstart(k, b)
                compute(b)
                dswait(b)
                sstart(b)
                sswait(b)
                offset_src(b)
                gstart(k + NB, b)
            return carry
        lax.fori_loop(1, GROUPS2 - 1, group, 0)
        for b in range(NB):
            k = (GROUPS2 - 1) * NB + b
            gwait(b)
            swait(b)
            dsstart(k, b)
            compute(b)
            dswait(b)
            sstart(b)
        for b in range(NB):
            swait(b)
        plsc.subcore_barrier()

        for k in range(COPIES):
            r0 = s * ROWS_PER_TILE + k * CH
            pltpu.sync_copy(acc.at[pl.ds(r0, CH)], o_l[0])
            pltpu.sync_copy(o_l[0], out_ref.at[c, pl.ds(r0, CH)])

    mesh = plsc.VectorSubcoreMesh(core_axis_name="c", subcore_axis_name="s")
    return pl.kernel(
        body,
        out_type=jax.ShapeDtypeStruct((NCORES, NPAD, p_row), jnp.float32),
        mesh=mesh,
        compiler_params=pltpu.CompilerParams(use_tc_tiling_on_sc=False,
                                             needs_layout_passes=False),
        scratch_types=[
            [pltpu.VMEM((CH,), jnp.int32)] * NB,
            pltpu.VMEM((CHUNKS2, CH), jnp.int32),
            [pltpu.VMEM((CH,), jnp.int32)] * NB,
            [pltpu.VMEM((CH, p_row), jnp.float32)] * NB,
            [pltpu.VMEM((CH, 16), jnp.float32)] * NB,
            [pltpu.VMEM((CH, p_row), jnp.float32)] * NB,
            pltpu.VMEM_SHARED((NPAD, p_row), jnp.float32),
            [pltpu.SemaphoreType.DMA] * NB,
            [pltpu.SemaphoreType.DMA] * NB,
            [pltpu.SemaphoreType.DMA] * NB,
            [pltpu.SemaphoreType.DMA] * NB,
            [pltpu.SemaphoreType.DMA] * NB,
        ],
    )


_edge_pass_l1 = _make_edge_pass_l1()
_edge_pass_2 = _make_edge_pass(32, 16, 1)


def kernel(edge_index, h, snorm_n, snorm_e, W1, attn_l1, attn_r1,
           W2, attn_l2, attn_r2):
    pad = jnp.full((EP - E,), N, jnp.int32)
    src = jnp.concatenate([edge_index[0], pad]).reshape(NW, CHUNKS, CH)
    dst = jnp.concatenate([edge_index[1], pad]).reshape(NW, CHUNKS, CH)
    hp = jnp.pad(h, ((0, NPAD - N), (0, 0)))
    alf = attn_l1.reshape(1, H1 * HID)
    arf = attn_r1.reshape(1, H1 * HID)
    F1a, ER1a, F1b, ER1b = pl.pallas_call(
        _tc_prep1,
        grid=(GRID,),
        in_specs=[
            pl.BlockSpec((BLK, IN_DIM), lambda i: (i, 0)),
            pl.BlockSpec((IN_DIM, H1 * HID), lambda i: (0, 0)),
            pl.BlockSpec((1, H1 * HID), lambda i: (0, 0)),
            pl.BlockSpec((1, H1 * HID), lambda i: (0, 0)),
        ],
        out_specs=[
            pl.BlockSpec((BLK, 80), lambda i: (i, 0)),
            pl.BlockSpec((BLK, 16), lambda i: (i, 0)),
            pl.BlockSpec((BLK, 80), lambda i: (i, 0)),
            pl.BlockSpec((BLK, 16), lambda i: (i, 0)),
        ],
        out_shape=[
            jax.ShapeDtypeStruct((NPAD, 80), jnp.float32),
            jax.ShapeDtypeStruct((NPAD, 16), jnp.float32),
            jax.ShapeDtypeStruct((NPAD, 80), jnp.float32),
            jax.ShapeDtypeStruct((NPAD, 16), jnp.float32),
        ],
    )(hp, W1, alf, arf)

    src_t = src.reshape(NSUB, CHUNKS2, CH)
    dst_t = dst.reshape(NSUB, CHUNKS2, CH)
    F12 = jnp.concatenate([F1a, F1b], axis=0)
    ER12 = jnp.concatenate([ER1a, ER1b], axis=0)
    P1m = _edge_pass_l1(src_t, dst_t, F12, ER12)

    F2, ER2 = pl.pallas_call(
        _tc_mid,
        grid=(GRID,),
        in_specs=[
            pl.BlockSpec((NCORES, BLK, 80), lambda i: (0, i, 0)),
            pl.BlockSpec((H1 * HID, OUT), lambda i: (0, 0)),
            pl.BlockSpec((OUT, 1), lambda i: (0, 0)),
            pl.BlockSpec((OUT, 1), lambda i: (0, 0)),
        ],
        out_specs=[
            pl.BlockSpec((BLK, 32), lambda i: (i, 0)),
            pl.BlockSpec((BLK, 16), lambda i: (i, 0)),
        ],
        out_shape=[
            jax.ShapeDtypeStruct((NPAD, 32), jnp.float32),
            jax.ShapeDtypeStruct((NPAD, 16), jnp.float32),
        ],
    )(P1m, W2, attn_l2.reshape(OUT, 1), attn_r2.reshape(OUT, 1))

    P2 = _edge_pass_2(src, dst, F2, ER2)

    out = pl.pallas_call(
        _tc_fin,
        grid=(GRID,),
        in_specs=[pl.BlockSpec((NCORES, BLK, 32), lambda i: (0, i, 0))],
        out_specs=pl.BlockSpec((BLK, OUT), lambda i: (i, 0)),
        out_shape=jax.ShapeDtypeStruct((NPAD, OUT), jnp.float32),
    )(P2)
    return out[:N]
